# baseline (speedup 1.0000x reference)
import jax
import jax.numpy as jnp
from jax.experimental import pallas as pl
from jax.experimental.pallas import tpu as pltpu

_BS = 8
_SEQ = 2048
_FEAT = 1024
_OUT = 1024
_WIN = 512
_NW = _SEQ // _WIN
_SPAN = 528


def _wpool_kernel(j0_ref, idx_ref, score_ref, feat_ref, out_ref):
    b = pl.program_id(0)
    w = pl.program_id(1)

    @pl.when(w == 0)
    def _():
        out_ref[...] = jnp.zeros_like(out_ref)

    j0 = j0_ref[b, w]
    j0 = pl.multiple_of(j0, 8)
    rows = jax.lax.broadcasted_iota(jnp.int32, (_SPAN, 1), 0) + j0
    a = jnp.where(idx_ref[0, 0] == rows, score_ref[0, 0], 0.0)
    part = jax.lax.dot(a, feat_ref[0], preferred_element_type=jnp.float32)
    out_ref[0, pl.ds(j0, _SPAN), :] += part


def kernel(score, feature, out_len):
    s2 = score[:, :, 0]

    cumsum = jnp.cumsum(score, axis=1)
    cumsum = jnp.where(jnp.mod(cumsum, 1.0) < 0.01, cumsum - 0.01, cumsum)
    int_cumsum = jnp.floor(cumsum).astype(jnp.int32)
    int_cumsum = jnp.clip(int_cumsum, 0, out_len - 1)
    idx = int_cumsum[:, :, 0]

    idx4 = idx.reshape(_BS, _NW, 1, _WIN)
    s4 = s2.reshape(_BS, _NW, 1, _WIN)
    j0s = jnp.minimum((idx[:, :: _WIN] // 8) * 8, _OUT - _SPAN)

    grid_spec = pltpu.PrefetchScalarGridSpec(
        num_scalar_prefetch=1,
        grid=(_BS, _NW),
        in_specs=[
            pl.BlockSpec((1, 1, 1, _WIN), lambda b, w, j0r: (b, w, 0, 0)),
            pl.BlockSpec((1, 1, 1, _WIN), lambda b, w, j0r: (b, w, 0, 0)),
            pl.BlockSpec((1, _WIN, _FEAT), lambda b, w, j0r: (b, w, 0)),
        ],
        out_specs=pl.BlockSpec((1, _OUT, _FEAT), lambda b, w, j0r: (b, 0, 0)),
    )
    out = pl.pallas_call(
        _wpool_kernel,
        grid_spec=grid_spec,
        out_shape=jax.ShapeDtypeStruct((_BS, _OUT, _FEAT), jnp.float32),
    )(j0s, idx4, s4, feature)
    return out


# speedup vs baseline: 1.0239x; 1.0239x over previous
import jax
import jax.numpy as jnp
from jax.experimental import pallas as pl
from jax.experimental.pallas import tpu as pltpu

_BS = 8
_SEQ = 2048
_FEAT = 1024
_OUT = 1024
_WIN = 512
_NW = _SEQ // _WIN
_SPAN = 528


def _wpool_kernel(idx_ref, score_ref, feat_ref, out_ref):
    w = pl.program_id(1)

    @pl.when(w == 0)
    def _():
        out_ref[...] = jnp.zeros_like(out_ref)

    v0 = idx_ref[0, 0, 0, 0]
    j0 = jnp.minimum((v0 // 8) * 8, _OUT - _SPAN)
    j0 = pl.multiple_of(j0, 8)
    rows = jax.lax.broadcasted_iota(jnp.int32, (_SPAN, 1), 0) + j0
    a = jnp.where(idx_ref[0, 0] == rows, score_ref[0, 0], 0.0)
    part = jax.lax.dot(a, feat_ref[0], preferred_element_type=jnp.float32)
    out_ref[0, pl.ds(j0, _SPAN), :] += part


def kernel(score, feature, out_len):
    s2 = score[:, :, 0]

    cumsum = jnp.cumsum(score, axis=1)
    cumsum = jnp.where(jnp.mod(cumsum, 1.0) < 0.01, cumsum - 0.01, cumsum)
    int_cumsum = jnp.floor(cumsum).astype(jnp.int32)
    int_cumsum = jnp.clip(int_cumsum, 0, out_len - 1)
    idx = int_cumsum[:, :, 0]

    idx4 = idx.reshape(_BS, _NW, 1, _WIN)
    s4 = s2.reshape(_BS, _NW, 1, _WIN)

    out = pl.pallas_call(
        _wpool_kernel,
        grid=(_BS, _NW),
        in_specs=[
            pl.BlockSpec((1, 1, 1, _WIN), lambda b, w: (b, w, 0, 0)),
            pl.BlockSpec((1, 1, 1, _WIN), lambda b, w: (b, w, 0, 0)),
            pl.BlockSpec((1, _WIN, _FEAT), lambda b, w: (b, w, 0)),
        ],
        out_specs=pl.BlockSpec((1, _OUT, _FEAT), lambda b, w: (b, 0, 0)),
        out_shape=jax.ShapeDtypeStruct((_BS, _OUT, _FEAT), jnp.float32),
    )(idx4, s4, feature)
    return out
